# Initial kernel scaffold; baseline (speedup 1.0000x reference)
import jax
import jax.numpy as jnp
from jax.experimental import pallas as pl


def kernel(r_node, i_node, r_edge, d_edge, edge_index, r2d_dst, d_segment_ids, W_rnode, W_inode, W_redge, W_msg, W_self, W_out, ln_gamma, ln_beta, lin_w, lin_b):
    raise NotImplementedError("write your pallas kernel here")


# speedup vs baseline: 1.9508x; 1.9508x over previous
import functools

import jax
import jax.numpy as jnp
from jax import lax
from jax.experimental import pallas as pl
from jax.experimental.pallas import tpu as pltpu
from jax.experimental.pallas import tpu_sc as plsc

N = 10000
E = 320000
D_NODE = 128
D_EDGE = 16
H = 64
L = 4
B = 16

NC, NS = 2, 16
NW = NC * NS
NPAD = 10240
E_PAD = 327680
EW = E_PAD // NW
CE = 128
NCH = EW // CE
RPT = NPAD // NS
CP = NPAD // NW

_mesh = plsc.VectorSubcoreMesh(core_axis_name="c", subcore_axis_name="s")


def _leaky(x):
    return jnp.maximum(x, 0.0) + 0.01 * jnp.minimum(x, 0.0)


def _embed_body(rn_ref, in_ref, wr_ref, wi_ref, wm_ref, h_ref, p_ref):
    h = jnp.dot(rn_ref[...], wr_ref[...], preferred_element_type=jnp.float32)
    h = h + in_ref[...] * wi_ref[0]
    h_ref[...] = h
    p_ref[...] = jnp.dot(h, wm_ref[...], preferred_element_type=jnp.float32)


_embed = pl.pallas_call(
    _embed_body,
    grid=(8,),
    in_specs=[
        pl.BlockSpec((1280, D_NODE), lambda i: (i, 0)),
        pl.BlockSpec((1280, 1), lambda i: (i, 0)),
        pl.BlockSpec((D_NODE, H), lambda i: (0, 0)),
        pl.BlockSpec((1, H), lambda i: (0, 0)),
        pl.BlockSpec((H, H), lambda i: (0, 0)),
    ],
    out_specs=[
        pl.BlockSpec((1280, H), lambda i: (i, 0)),
        pl.BlockSpec((1280, H), lambda i: (i, 0)),
    ],
    out_shape=[
        jax.ShapeDtypeStruct((NPAD, H), jnp.float32),
        jax.ShapeDtypeStruct((NPAD, H), jnp.float32),
    ],
)


def _q_body(re_ref, wre_ref, wm_ref, q_ref):
    re = re_ref[...]
    for l in range(L):
        wq = jnp.dot(wre_ref[...], wm_ref[l], preferred_element_type=jnp.float32)
        q_ref[l] = jnp.dot(re, wq, preferred_element_type=jnp.float32)


_qcall = pl.pallas_call(
    _q_body,
    grid=(E_PAD // 2048,),
    in_specs=[
        pl.BlockSpec((2048, D_EDGE), lambda i: (i, 0)),
        pl.BlockSpec((D_EDGE, H), lambda i: (0, 0)),
        pl.BlockSpec((L, H, H), lambda i: (0, 0, 0)),
    ],
    out_specs=pl.BlockSpec((L, 2048, H), lambda i: (0, i, 0)),
    out_shape=jax.ShapeDtypeStruct((L, E_PAD, H), jnp.float32),
)


def _update_mid_body(ag_ref, h_ref, ws_ref, wm_ref, hn_ref, p_ref):
    agg = ag_ref[0] + ag_ref[1]
    u = jnp.dot(agg, ws_ref[...], preferred_element_type=jnp.float32)
    hn = h_ref[...] + 0.1 * _leaky(u)
    hn_ref[...] = hn
    p_ref[...] = jnp.dot(hn, wm_ref[...], preferred_element_type=jnp.float32)


_update_mid = pl.pallas_call(
    _update_mid_body,
    grid=(8,),
    in_specs=[
        pl.BlockSpec((2, 1280, H), lambda i: (0, i, 0)),
        pl.BlockSpec((1280, H), lambda i: (i, 0)),
        pl.BlockSpec((H, H), lambda i: (0, 0)),
        pl.BlockSpec((H, H), lambda i: (0, 0)),
    ],
    out_specs=[
        pl.BlockSpec((1280, H), lambda i: (i, 0)),
        pl.BlockSpec((1280, H), lambda i: (i, 0)),
    ],
    out_shape=[
        jax.ShapeDtypeStruct((NPAD, H), jnp.float32),
        jax.ShapeDtypeStruct((NPAD, H), jnp.float32),
    ],
)


def _update_last_body(ag_ref, h_ref, ws_ref, de_ref, wo_ref, s_ref):
    agg = ag_ref[0] + ag_ref[1]
    u = jnp.dot(agg, ws_ref[...], preferred_element_type=jnp.float32)
    hn = h_ref[...] + 0.1 * _leaky(u)
    s_ref[...] = jnp.dot(hn * de_ref[...], wo_ref[...],
                         preferred_element_type=jnp.float32)


_update_last = pl.pallas_call(
    _update_last_body,
    grid=(8,),
    in_specs=[
        pl.BlockSpec((2, 1280, H), lambda i: (0, i, 0)),
        pl.BlockSpec((1280, H), lambda i: (i, 0)),
        pl.BlockSpec((H, H), lambda i: (0, 0)),
        pl.BlockSpec((1280, 1), lambda i: (i, 0)),
        pl.BlockSpec((H, 1), lambda i: (0, 0)),
    ],
    out_specs=pl.BlockSpec((1280, 1), lambda i: (i, 0)),
    out_shape=jax.ShapeDtypeStruct((NPAD, 1), jnp.float32),
)


def _edge_body(p_hbm, q_hbm, src_hbm, dst_hbm, out_hbm,
               srcv, dstv, prow, qrow, zrow, aggsh, sem_g):
    c = lax.axis_index("c")
    s = lax.axis_index("s")
    wid = s * NC + c

    def _zb(i, _):
        for j in range(H // 16):
            zrow[i, pl.ds(j * 16, 16)] = jnp.zeros((16,), jnp.float32)
        return _
    lax.fori_loop(0, CE, _zb, None)

    def _zs(t, _):
        pltpu.sync_copy(zrow, aggsh.at[pl.ds(s * RPT + t * CE, CE)])
        return _
    lax.fori_loop(0, RPT // CE, _zs, None)
    plsc.subcore_barrier()

    e_base = wid * EW

    def _chunk(k, _):
        e0 = e_base + k * CE
        pltpu.sync_copy(src_hbm.at[pl.ds(e0, CE)], srcv.at[0])
        pltpu.sync_copy(dst_hbm.at[pl.ds(e0, CE)], dstv.at[0])
        pltpu.sync_copy(q_hbm.at[pl.ds(e0, CE)], qrow.at[0])
        pltpu.async_copy(p_hbm.at[srcv.at[0]], prow.at[0], sem_g).wait()

        def _crow(r, _c):
            for j in range(H // 16):
                x = (prow[0, r, pl.ds(j * 16, 16)]
                     + qrow[0, r, pl.ds(j * 16, 16)])
                prow[0, r, pl.ds(j * 16, 16)] = _leaky(x)
            return _c
        lax.fori_loop(0, CE, _crow, None)
        pltpu.sync_copy(prow.at[0], aggsh.at[dstv.at[0]], add=True)
        return _
    lax.fori_loop(0, NCH, _chunk, None)

    plsc.subcore_barrier()
    pltpu.sync_copy(aggsh.at[pl.ds(s * RPT, RPT)],
                    out_hbm.at[c, pl.ds(s * RPT, RPT)])


_edge = functools.partial(
    pl.kernel,
    out_type=jax.ShapeDtypeStruct((NC, NPAD, H), jnp.float32),
    mesh=_mesh,
    compiler_params=pltpu.CompilerParams(use_tc_tiling_on_sc=False),
    scratch_types=[
        pltpu.VMEM((1, CE), jnp.int32),
        pltpu.VMEM((1, CE), jnp.int32),
        pltpu.VMEM((1, CE, H), jnp.float32),
        pltpu.VMEM((1, CE, H), jnp.float32),
        pltpu.VMEM((CE, H), jnp.float32),
        pltpu.VMEM_SHARED((NPAD, H), jnp.float32),
        pltpu.SemaphoreType.DMA,
    ],
)(_edge_body)


def _pool_body(seg_hbm, r2d_hbm, s_hbm, out_hbm, segv, idxv, sv, bins):
    c = lax.axis_index("c")
    s_ = lax.axis_index("s")
    wid = s_ * NC + c
    pltpu.sync_copy(seg_hbm, segv)
    pltpu.sync_copy(r2d_hbm.at[pl.ds(wid * CP, CP)], idxv)
    pltpu.sync_copy(s_hbm.at[pl.ds(wid * CP, CP)], sv)
    for i in range(B):
        bins[i] = jnp.zeros((16,), jnp.float32)
    lane = lax.iota(jnp.int32, 16)

    def _st(i, _):
        iv = idxv[pl.ds(i * 16, 16)]
        vals = sv[pl.ds(i * 16, 16)]
        gj = wid * CP + i * 16 + lane
        segs = plsc.load_gather(segv, [iv])
        plsc.addupdate_scatter(bins, [segs, lane], vals, mask=gj < N)
        return _
    lax.fori_loop(0, CP // 16, _st, None)
    pltpu.sync_copy(bins, out_hbm.at[wid])


_pool = functools.partial(
    pl.kernel,
    out_type=jax.ShapeDtypeStruct((NW, B, 16), jnp.float32),
    mesh=_mesh,
    compiler_params=pltpu.CompilerParams(use_tc_tiling_on_sc=False,
                                         needs_layout_passes=False),
    scratch_types=[
        pltpu.VMEM((NPAD,), jnp.int32),
        pltpu.VMEM((CP,), jnp.int32),
        pltpu.VMEM((CP,), jnp.float32),
        pltpu.VMEM((B, 16), jnp.float32),
    ],
)(_pool_body)


def _final_body(part_ref, g_ref, b_ref, lw_ref, lb_ref, o_ref):
    pooled = jnp.sum(jnp.sum(part_ref[...], axis=0), axis=1, keepdims=True)
    mu = jnp.mean(pooled, axis=-1, keepdims=True)
    var = jnp.mean((pooled - mu) ** 2, axis=-1, keepdims=True)
    normed = (pooled - mu) / jnp.sqrt(var + 1e-5) * g_ref[0, 0] + b_ref[0, 0]
    o_ref[...] = jnp.tanh(normed) * lw_ref[0, 0] + lb_ref[0, 0]


_final = pl.pallas_call(
    _final_body,
    out_shape=jax.ShapeDtypeStruct((B, 1), jnp.float32),
)


def kernel(r_node, i_node, r_edge, d_edge, edge_index, r2d_dst, d_segment_ids,
           W_rnode, W_inode, W_redge, W_msg, W_self, W_out,
           ln_gamma, ln_beta, lin_w, lin_b):
    f32 = jnp.float32
    pe = E_PAD - E
    pn = NPAD - N
    srcp = jnp.concatenate([edge_index[0], jnp.zeros((pe,), jnp.int32)])
    dstp = jnp.concatenate([edge_index[1], jnp.full((pe,), N, jnp.int32)])
    rep = jnp.pad(r_edge.astype(f32), ((0, pe), (0, 0)))
    rnp = jnp.pad(r_node.astype(f32), ((0, pn), (0, 0)))
    inp = jnp.pad(i_node.astype(f32), ((0, pn), (0, 0)))
    dep = jnp.pad(d_edge.astype(f32), ((0, pn), (0, 0)))
    r2dp = jnp.pad(r2d_dst, (0, pn))
    segp = jnp.pad(d_segment_ids, (0, pn))

    h, P = _embed(rnp, inp, W_rnode, W_inode, W_msg[0])
    Q = _qcall(rep, W_redge, W_msg)
    for l in range(L):
        aggp = _edge(P, Q[l], srcp, dstp)
        if l < L - 1:
            h, P = _update_mid(aggp, h, W_self[l], W_msg[l + 1])
        else:
            s = _update_last(aggp, h, W_self[l], dep, W_out)
    part = _pool(segp, r2dp, s.reshape(NPAD))
    return _final(part, ln_gamma.reshape(1, 1), ln_beta.reshape(1, 1),
                  lin_w, lin_b.reshape(1, 1))
